# baseline (speedup 1.0000x reference)
import functools

import jax
import jax.numpy as jnp
from jax.experimental import pallas as pl
from jax.experimental.pallas import tpu as pltpu

N_TOKENS = 16384
D_MODEL = 2048
NUM_EXPERTS = 16
TOP_K = 2
BLOCK_T = 1024
N_SUB = 8
SUB_T = BLOCK_T // N_SUB


def _router_kernel(*refs):
    x_refs = refs[:N_SUB]
    w_ref, b_ref, rw_ref, idx_ref, probs_ref = refs[N_SUB:]
    w = w_ref[...]
    b = b_ref[...]
    for j in range(N_SUB):
        x = x_refs[j][...]
        logits = jax.lax.dot_general(
            x, w, (((1,), (1,)), ((), ())), preferred_element_type=jnp.float32
        ) + b

        m = jnp.max(logits, axis=1, keepdims=True)
        e = jnp.exp(logits - m)
        probs_ref[pl.ds(j * SUB_T, SUB_T), :] = e / jnp.sum(e, axis=1, keepdims=True)

        col = jax.lax.broadcasted_iota(jnp.int32, logits.shape, 1)
        big = jnp.int32(NUM_EXPERTS)

        idx0 = jnp.min(jnp.where(logits == m, col, big), axis=1, keepdims=True)

        neg = jnp.float32(-jnp.inf)
        l1 = jnp.where(col == idx0, neg, logits)
        v1 = jnp.max(l1, axis=1, keepdims=True)
        idx1 = jnp.min(jnp.where(l1 == v1, col, big), axis=1, keepdims=True)

        p1 = jax.nn.sigmoid(v1 - m)
        p0 = 1.0 - p1

        rw_ref[pl.ds(j * SUB_T, SUB_T), :] = (
            jnp.where(col == idx0, p0, 0.0) + jnp.where(col == idx1, p1, 0.0)
        )
        idx_ref[pl.ds(j * SUB_T, SUB_T), :] = jnp.concatenate([idx0, idx1], axis=1)


def _x_spec(j):
    return pl.BlockSpec((SUB_T, D_MODEL), lambda i, j=j: (i * N_SUB + j, 0))


@functools.partial(jax.jit, static_argnames=())
def kernel(x, W, b):
    grid = (N_TOKENS // BLOCK_T,)
    rw, idx, probs = pl.pallas_call(
        _router_kernel,
        grid=grid,
        in_specs=[_x_spec(j) for j in range(N_SUB)] + [
            pl.BlockSpec((NUM_EXPERTS, D_MODEL), lambda i: (0, 0)),
            pl.BlockSpec((1, NUM_EXPERTS), lambda i: (0, 0)),
        ],
        out_specs=[
            pl.BlockSpec((BLOCK_T, NUM_EXPERTS), lambda i: (i, 0)),
            pl.BlockSpec((BLOCK_T, TOP_K), lambda i: (i, 0)),
            pl.BlockSpec((BLOCK_T, NUM_EXPERTS), lambda i: (i, 0)),
        ],
        out_shape=[
            jax.ShapeDtypeStruct((N_TOKENS, NUM_EXPERTS), jnp.float32),
            jax.ShapeDtypeStruct((N_TOKENS, TOP_K), jnp.int32),
            jax.ShapeDtypeStruct((N_TOKENS, NUM_EXPERTS), jnp.float32),
        ],
        compiler_params=pltpu.CompilerParams(
            dimension_semantics=("parallel",),
        ),
    )(*([x] * N_SUB), W, b.reshape(1, NUM_EXPERTS))
    return rw, idx, probs


# speedup vs baseline: 1.0412x; 1.0412x over previous
import functools

import jax
import jax.numpy as jnp
from jax.experimental import pallas as pl
from jax.experimental.pallas import tpu as pltpu

N_TOKENS = 16384
D_MODEL = 2048
NUM_EXPERTS = 16
TOP_K = 2
BLOCK_T = 2048
N_SUB = 4
SUB_T = BLOCK_T // N_SUB


def _router_kernel(*refs):
    x_refs = refs[:N_SUB]
    w_ref, b_ref, rw_ref, idx_ref, probs_ref = refs[N_SUB:]
    w = w_ref[...]
    b = b_ref[...]
    for j in range(N_SUB):
        x = x_refs[j][...]
        logits = jax.lax.dot_general(
            x, w, (((1,), (1,)), ((), ())), preferred_element_type=jnp.float32
        ) + b

        m = jnp.max(logits, axis=1, keepdims=True)
        e = jnp.exp(logits - m)
        probs_ref[pl.ds(j * SUB_T, SUB_T), :] = e / jnp.sum(e, axis=1, keepdims=True)

        col = jax.lax.broadcasted_iota(jnp.int32, logits.shape, 1)
        big = jnp.int32(NUM_EXPERTS)

        idx0 = jnp.min(jnp.where(logits == m, col, big), axis=1, keepdims=True)

        neg = jnp.float32(-jnp.inf)
        l1 = jnp.where(col == idx0, neg, logits)
        v1 = jnp.max(l1, axis=1, keepdims=True)
        idx1 = jnp.min(jnp.where(l1 == v1, col, big), axis=1, keepdims=True)

        p1 = jax.nn.sigmoid(v1 - m)
        p0 = 1.0 - p1

        rw_ref[pl.ds(j * SUB_T, SUB_T), :] = (
            jnp.where(col == idx0, p0, 0.0) + jnp.where(col == idx1, p1, 0.0)
        )
        idx_ref[pl.ds(j * SUB_T, SUB_T), :] = jnp.concatenate([idx0, idx1], axis=1)


def _x_spec(j):
    return pl.BlockSpec((SUB_T, D_MODEL), lambda i, j=j: (i * N_SUB + j, 0))


@functools.partial(jax.jit, static_argnames=())
def kernel(x, W, b):
    grid = (N_TOKENS // BLOCK_T,)
    rw, idx, probs = pl.pallas_call(
        _router_kernel,
        grid=grid,
        in_specs=[_x_spec(j) for j in range(N_SUB)] + [
            pl.BlockSpec((NUM_EXPERTS, D_MODEL), lambda i: (0, 0)),
            pl.BlockSpec((1, NUM_EXPERTS), lambda i: (0, 0)),
        ],
        out_specs=[
            pl.BlockSpec((BLOCK_T, NUM_EXPERTS), lambda i: (i, 0)),
            pl.BlockSpec((BLOCK_T, TOP_K), lambda i: (i, 0)),
            pl.BlockSpec((BLOCK_T, NUM_EXPERTS), lambda i: (i, 0)),
        ],
        out_shape=[
            jax.ShapeDtypeStruct((N_TOKENS, NUM_EXPERTS), jnp.float32),
            jax.ShapeDtypeStruct((N_TOKENS, TOP_K), jnp.int32),
            jax.ShapeDtypeStruct((N_TOKENS, NUM_EXPERTS), jnp.float32),
        ],
        compiler_params=pltpu.CompilerParams(
            dimension_semantics=("parallel",),
        ),
    )(*([x] * N_SUB), W, b.reshape(1, NUM_EXPERTS))
    return rw, idx, probs
